# baseline (speedup 1.0000x reference)
import functools

import jax
import jax.numpy as jnp
from jax import lax
from jax.experimental import pallas as pl
from jax.experimental.pallas import tpu as pltpu
from jax.experimental.pallas import tpu_sc as plsc

_N_EDGES = 160000
_N = 2 * _N_EDGES
_NW = 16
_NV = 2 * _NW
_S = _N // _NW
_SV = _N // _NV
_NB = 10240
_BR = _NB // _NW
_CHUNK = 128
_NCH = _SV // _CHUNK


def _body(src_hbm, dst_hbm, out_hbm, keys_v, vals_v, hist_a, hist_b,
          block_v, loc_v, acc_v, tots_v, pos_a0, pos_a1, pos_b0, pos_b1,
          pos_ta, pos_tb, carry_s, hist_all_s, totals_s, sem_k, sem_v,
          sem_h, sem_a0, sem_a1, sem_b0, sem_b1):
  wid = lax.axis_index("s")
  zeros = jnp.zeros((16,), jnp.int32)

  off = (wid % 8) * _S

  @pl.when(wid < 8)
  def _():
    pltpu.async_copy(src_hbm.at[pl.ds(off, _S)], keys_v, sem_k)
    pltpu.async_copy(dst_hbm.at[pl.ds(off, _S)], vals_v.at[pl.ds(0, _S)],
                     sem_v)

  @pl.when(wid >= 8)
  def _():
    pltpu.async_copy(dst_hbm.at[pl.ds(off, _S)], keys_v, sem_k)
    pltpu.async_copy(src_hbm.at[pl.ds(off, _S)], vals_v.at[pl.ds(0, _S)],
                     sem_v)

  @pl.loop(0, _NB // 16)
  def _(i):
    hist_a[pl.ds(i * 16, 16)] = zeros
    hist_b[pl.ds(i * 16, 16)] = zeros

  pltpu.make_async_copy(src_hbm.at[pl.ds(off, _S)], keys_v, sem_k).wait()

  @pl.loop(0, _SV // 16)
  def _(i):
    ka = keys_v[pl.ds(i * 16, 16)]
    kb = keys_v[pl.ds(_SV + i * 16, 16)]
    run_a, last_a = plsc.scan_count(ka)
    run_b, last_b = plsc.scan_count(kb)
    cur_a = plsc.load_gather(hist_a, [ka])
    cur_b = plsc.load_gather(hist_b, [kb])
    keys_v[pl.ds(i * 16, 16)] = ka + ((cur_a + run_a - 1) << 14)
    keys_v[pl.ds(_SV + i * 16, 16)] = kb + ((cur_b + run_b - 1) << 14)
    plsc.addupdate_scatter(hist_a, [ka], run_a, mask=last_a)
    plsc.addupdate_scatter(hist_b, [kb], run_b, mask=last_b)

  pltpu.async_copy(hist_a, hist_all_s.at[pl.ds((2 * wid) * _NB, _NB)], sem_h)
  pltpu.async_copy(hist_b, hist_all_s.at[pl.ds((2 * wid + 1) * _NB, _NB)],
                   sem_h)
  pltpu.make_async_copy(hist_a, hist_all_s.at[pl.ds((2 * wid) * _NB, _NB)],
                        sem_h).wait()
  pltpu.make_async_copy(hist_b, hist_all_s.at[pl.ds((2 * wid) * _NB, _NB)],
                        sem_h).wait()
  plsc.subcore_barrier()

  for v2 in range(_NV):
    pltpu.async_copy(hist_all_s.at[pl.ds(v2 * _NB + wid * _BR, _BR)],
                     block_v.at[pl.ds(v2 * _BR, _BR)], sem_h)
  for v2 in range(_NV):
    pltpu.make_async_copy(hist_all_s.at[pl.ds(v2 * _NB + wid * _BR, _BR)],
                          block_v.at[pl.ds(v2 * _BR, _BR)], sem_h).wait()

  @pl.loop(0, _BR // 16)
  def _(g):
    acc_v[pl.ds(g * 16, 16)] = zeros

  for v2 in range(_NV):

    @pl.loop(0, _BR // 16)
    def _(g, v2=v2):
      a = acc_v[pl.ds(g * 16, 16)]
      h = block_v[pl.ds(v2 * _BR + g * 16, 16)]
      block_v[pl.ds(v2 * _BR + g * 16, 16)] = a
      acc_v[pl.ds(g * 16, 16)] = a + h

  carry_s[0] = 0

  @pl.loop(0, _BR // 16)
  def _(g):
    v = acc_v[pl.ds(g * 16, 16)]
    c = plsc.cumsum(v)
    cin = carry_s[0]
    loc_v[pl.ds(g * 16, 16)] = c - v + cin
    carry_s[0] = cin + jnp.sum(v)

  tots_v[...] = jnp.full((16,), carry_s[0], jnp.int32)
  pltpu.sync_copy(tots_v, totals_s.at[pl.ds(wid * 16, 16)])
  plsc.subcore_barrier()

  pvec = zeros
  for w2 in range(_NW):
    pltpu.sync_copy(totals_s.at[pl.ds(w2 * 16, 16)], tots_v)
    gate = jnp.where(w2 < wid, 1, 0).astype(jnp.int32)
    pvec = pvec + tots_v[...] * gate

  for v2 in range(_NV):

    @pl.loop(0, _BR // 16)
    def _(g, v2=v2, pvec=pvec):
      o = v2 * _BR + g * 16
      block_v[pl.ds(o, 16)] = (block_v[pl.ds(o, 16)] +
                               loc_v[pl.ds(g * 16, 16)] + pvec)

    pltpu.async_copy(block_v.at[pl.ds(v2 * _BR, _BR)],
                     hist_all_s.at[pl.ds(v2 * _NB + wid * _BR, _BR)], sem_h)
  for v2 in range(_NV):
    pltpu.make_async_copy(block_v.at[pl.ds(v2 * _BR, _BR)],
                          hist_all_s.at[pl.ds(v2 * _NB + wid * _BR, _BR)],
                          sem_h).wait()
  plsc.subcore_barrier()

  pltpu.async_copy(hist_all_s.at[pl.ds((2 * wid) * _NB, _NB)], hist_a, sem_h)
  pltpu.async_copy(hist_all_s.at[pl.ds((2 * wid + 1) * _NB, _NB)], hist_b,
                   sem_h)
  pltpu.make_async_copy(hist_all_s.at[pl.ds((2 * wid) * _NB, _NB)], hist_a,
                        sem_h).wait()
  pltpu.make_async_copy(hist_all_s.at[pl.ds((2 * wid) * _NB, _NB)], hist_b,
                        sem_h).wait()
  pltpu.make_async_copy(dst_hbm.at[pl.ds(off, _S)], vals_v.at[pl.ds(0, _S)],
                        sem_v).wait()

  mask14 = jnp.full((16,), (1 << 14) - 1, jnp.int32)

  def _chunk(c, pos_a, pos_b):
    for j in range(_CHUNK // 16):
      pa = keys_v[pl.ds(c * _CHUNK + j * 16, 16)]
      pb = keys_v[pl.ds(_SV + c * _CHUNK + j * 16, 16)]
      pos_a[pl.ds(j * 16, 16)] = (
          plsc.load_gather(hist_a, [pa & mask14]) + (pa >> 14))
      pos_b[pl.ds(j * 16, 16)] = (
          plsc.load_gather(hist_b, [pb & mask14]) + (pb >> 14))

  def _fire(c, pos_a, pos_b, sa, sb):
    pltpu.async_copy(vals_v.at[pl.ds(c * _CHUNK, _CHUNK)],
                     out_hbm.at[pos_a], sa)
    pltpu.async_copy(vals_v.at[pl.ds(_SV + c * _CHUNK, _CHUNK)],
                     out_hbm.at[pos_b], sb)

  def _drain(c, pos_a, pos_b, sa, sb):
    pltpu.make_async_copy(vals_v.at[pl.ds(c * _CHUNK, _CHUNK)],
                          out_hbm.at[pos_a], sa).wait()
    pltpu.make_async_copy(vals_v.at[pl.ds(_SV + c * _CHUNK, _CHUNK)],
                          out_hbm.at[pos_b], sb).wait()

  @pl.loop(0, _NCH // 2)
  def _(h):
    c0 = 2 * h
    c1 = 2 * h + 1

    @pl.when(h > 0)
    def _():
      _drain(c0 - 2, pos_a0, pos_b0, sem_a0, sem_b0)

    _chunk(c0, pos_a0, pos_b0)
    _fire(c0, pos_a0, pos_b0, sem_a0, sem_b0)

    @pl.when(h > 0)
    def _():
      _drain(c1 - 2, pos_a1, pos_b1, sem_a1, sem_b1)

    _chunk(c1, pos_a1, pos_b1)
    _fire(c1, pos_a1, pos_b1, sem_a1, sem_b1)

  _drain(_NCH - 2, pos_a0, pos_b0, sem_a0, sem_b0)
  _drain(_NCH - 1, pos_a1, pos_b1, sem_a1, sem_b1)

  pa = keys_v[pl.ds(_NCH * _CHUNK, 16)]
  pb = keys_v[pl.ds(_SV + _NCH * _CHUNK, 16)]
  pos_ta[...] = plsc.load_gather(hist_a, [pa & mask14]) + (pa >> 14)
  pos_tb[...] = plsc.load_gather(hist_b, [pb & mask14]) + (pb >> 14)
  pltpu.sync_copy(vals_v.at[pl.ds(_NCH * _CHUNK, 16)], out_hbm.at[pos_ta])
  pltpu.sync_copy(vals_v.at[pl.ds(_SV + _NCH * _CHUNK, 16)],
                  out_hbm.at[pos_tb])


_sort = pl.kernel(
    _body,
    out_type=jax.ShapeDtypeStruct((_N,), jnp.int32),
    mesh=plsc.VectorSubcoreMesh(
        core_axis_name="c", subcore_axis_name="s", num_cores=1),
    compiler_params=pltpu.CompilerParams(needs_layout_passes=False),
    scratch_types=[
        pltpu.VMEM((_S,), jnp.int32),
        pltpu.VMEM((_S,), jnp.int32),
        pltpu.VMEM((_NB,), jnp.int32),
        pltpu.VMEM((_NB,), jnp.int32),
        pltpu.VMEM((_NV * _BR,), jnp.int32),
        pltpu.VMEM((_BR,), jnp.int32),
        pltpu.VMEM((_BR,), jnp.int32),
        pltpu.VMEM((16,), jnp.int32),
        pltpu.VMEM((_CHUNK,), jnp.int32),
        pltpu.VMEM((_CHUNK,), jnp.int32),
        pltpu.VMEM((_CHUNK,), jnp.int32),
        pltpu.VMEM((_CHUNK,), jnp.int32),
        pltpu.VMEM((16,), jnp.int32),
        pltpu.VMEM((16,), jnp.int32),
        pltpu.SMEM((1,), jnp.int32),
        pltpu.VMEM_SHARED((_NV * _NB,), jnp.int32),
        pltpu.VMEM_SHARED((_NW * 16,), jnp.int32),
        pltpu.SemaphoreType.DMA,
        pltpu.SemaphoreType.DMA,
        pltpu.SemaphoreType.DMA,
        pltpu.SemaphoreType.DMA,
        pltpu.SemaphoreType.DMA,
        pltpu.SemaphoreType.DMA,
        pltpu.SemaphoreType.DMA,
    ],
)


@jax.jit
def kernel(edges, nodes):
  e = edges.astype(jnp.int32)
  flat = _sort(e[:, 0], e[:, 1])
  return flat.reshape(nodes.shape[0], -1)


# speedup vs baseline: 5.5137x; 5.5137x over previous
import functools

import jax
import jax.numpy as jnp
from jax import lax
from jax.experimental import pallas as pl
from jax.experimental.pallas import tpu as pltpu
from jax.experimental.pallas import tpu_sc as plsc

_N_EDGES = 160000
_N = 2 * _N_EDGES
_NW = 16
_NV = 2 * _NW
_S = _N // _NW
_SV = _N // _NV
_NB = 10240
_BR = _NB // _NW
_CHUNK = 128
_NCH = _SV // _CHUNK


def _body(src_hbm, dst_hbm, out_hbm, keys_v, vals_v, hist_a, hist_b,
          block_v, loc_v, acc_v, tots_v, pos_a0, pos_a1, pos_b0, pos_b1,
          pos_ta, pos_tb, carry_s, hist_all_s, totals_s, out_s, sem_k,
          sem_v, sem_h, sem_a0, sem_a1, sem_b0, sem_b1):
  wid = lax.axis_index("s")
  zeros = jnp.zeros((16,), jnp.int32)

  off = (wid % 8) * _S

  @pl.when(wid < 8)
  def _():
    pltpu.async_copy(src_hbm.at[pl.ds(off, _S)], keys_v, sem_k)
    pltpu.async_copy(dst_hbm.at[pl.ds(off, _S)], vals_v.at[pl.ds(0, _S)],
                     sem_v)

  @pl.when(wid >= 8)
  def _():
    pltpu.async_copy(dst_hbm.at[pl.ds(off, _S)], keys_v, sem_k)
    pltpu.async_copy(src_hbm.at[pl.ds(off, _S)], vals_v.at[pl.ds(0, _S)],
                     sem_v)

  @pl.loop(0, _NB // 16)
  def _(i):
    hist_a[pl.ds(i * 16, 16)] = zeros
    hist_b[pl.ds(i * 16, 16)] = zeros

  pltpu.make_async_copy(src_hbm.at[pl.ds(off, _S)], keys_v, sem_k).wait()

  @pl.loop(0, _SV // 16)
  def _(i):
    ka = keys_v[pl.ds(i * 16, 16)]
    kb = keys_v[pl.ds(_SV + i * 16, 16)]
    run_a, last_a = plsc.scan_count(ka)
    run_b, last_b = plsc.scan_count(kb)
    cur_a = plsc.load_gather(hist_a, [ka])
    cur_b = plsc.load_gather(hist_b, [kb])
    keys_v[pl.ds(i * 16, 16)] = ka + ((cur_a + run_a - 1) << 14)
    keys_v[pl.ds(_SV + i * 16, 16)] = kb + ((cur_b + run_b - 1) << 14)
    plsc.addupdate_scatter(hist_a, [ka], run_a, mask=last_a)
    plsc.addupdate_scatter(hist_b, [kb], run_b, mask=last_b)

  pltpu.async_copy(hist_a, hist_all_s.at[pl.ds((2 * wid) * _NB, _NB)], sem_h)
  pltpu.async_copy(hist_b, hist_all_s.at[pl.ds((2 * wid + 1) * _NB, _NB)],
                   sem_h)
  pltpu.make_async_copy(hist_a, hist_all_s.at[pl.ds((2 * wid) * _NB, _NB)],
                        sem_h).wait()
  pltpu.make_async_copy(hist_b, hist_all_s.at[pl.ds((2 * wid) * _NB, _NB)],
                        sem_h).wait()
  plsc.subcore_barrier()

  for v2 in range(_NV):
    pltpu.async_copy(hist_all_s.at[pl.ds(v2 * _NB + wid * _BR, _BR)],
                     block_v.at[pl.ds(v2 * _BR, _BR)], sem_h)
  for v2 in range(_NV):
    pltpu.make_async_copy(hist_all_s.at[pl.ds(v2 * _NB + wid * _BR, _BR)],
                          block_v.at[pl.ds(v2 * _BR, _BR)], sem_h).wait()

  @pl.loop(0, _BR // 16)
  def _(g):
    acc_v[pl.ds(g * 16, 16)] = zeros

  for v2 in range(_NV):

    @pl.loop(0, _BR // 16)
    def _(g, v2=v2):
      a = acc_v[pl.ds(g * 16, 16)]
      h = block_v[pl.ds(v2 * _BR + g * 16, 16)]
      block_v[pl.ds(v2 * _BR + g * 16, 16)] = a
      acc_v[pl.ds(g * 16, 16)] = a + h

  carry_s[0] = 0

  @pl.loop(0, _BR // 16)
  def _(g):
    v = acc_v[pl.ds(g * 16, 16)]
    c = plsc.cumsum(v)
    cin = carry_s[0]
    loc_v[pl.ds(g * 16, 16)] = c - v + cin
    carry_s[0] = cin + jnp.sum(v)

  tots_v[...] = jnp.full((16,), carry_s[0], jnp.int32)
  pltpu.sync_copy(tots_v, totals_s.at[pl.ds(wid * 16, 16)])
  plsc.subcore_barrier()

  pvec = zeros
  for w2 in range(_NW):
    pltpu.sync_copy(totals_s.at[pl.ds(w2 * 16, 16)], tots_v)
    gate = jnp.where(w2 < wid, 1, 0).astype(jnp.int32)
    pvec = pvec + tots_v[...] * gate

  for v2 in range(_NV):

    @pl.loop(0, _BR // 16)
    def _(g, v2=v2, pvec=pvec):
      o = v2 * _BR + g * 16
      block_v[pl.ds(o, 16)] = (block_v[pl.ds(o, 16)] +
                               loc_v[pl.ds(g * 16, 16)] + pvec)

    pltpu.async_copy(block_v.at[pl.ds(v2 * _BR, _BR)],
                     hist_all_s.at[pl.ds(v2 * _NB + wid * _BR, _BR)], sem_h)
  for v2 in range(_NV):
    pltpu.make_async_copy(block_v.at[pl.ds(v2 * _BR, _BR)],
                          hist_all_s.at[pl.ds(v2 * _NB + wid * _BR, _BR)],
                          sem_h).wait()
  plsc.subcore_barrier()

  pltpu.async_copy(hist_all_s.at[pl.ds((2 * wid) * _NB, _NB)], hist_a, sem_h)
  pltpu.async_copy(hist_all_s.at[pl.ds((2 * wid + 1) * _NB, _NB)], hist_b,
                   sem_h)
  pltpu.make_async_copy(hist_all_s.at[pl.ds((2 * wid) * _NB, _NB)], hist_a,
                        sem_h).wait()
  pltpu.make_async_copy(hist_all_s.at[pl.ds((2 * wid) * _NB, _NB)], hist_b,
                        sem_h).wait()
  pltpu.make_async_copy(dst_hbm.at[pl.ds(off, _S)], vals_v.at[pl.ds(0, _S)],
                        sem_v).wait()

  mask14 = jnp.full((16,), (1 << 14) - 1, jnp.int32)

  def _chunk(c, pos_a, pos_b):
    for j in range(_CHUNK // 16):
      pa = keys_v[pl.ds(c * _CHUNK + j * 16, 16)]
      pb = keys_v[pl.ds(_SV + c * _CHUNK + j * 16, 16)]
      pos_a[pl.ds(j * 16, 16)] = (
          plsc.load_gather(hist_a, [pa & mask14]) + (pa >> 14))
      pos_b[pl.ds(j * 16, 16)] = (
          plsc.load_gather(hist_b, [pb & mask14]) + (pb >> 14))

  def _fire(c, pos_a, pos_b, sa, sb):
    pltpu.async_copy(vals_v.at[pl.ds(c * _CHUNK, _CHUNK)],
                     out_s.at[pos_a], sa)
    pltpu.async_copy(vals_v.at[pl.ds(_SV + c * _CHUNK, _CHUNK)],
                     out_s.at[pos_b], sb)

  def _drain(c, pos_a, pos_b, sa, sb):
    pltpu.make_async_copy(vals_v.at[pl.ds(c * _CHUNK, _CHUNK)],
                          out_s.at[pos_a], sa).wait()
    pltpu.make_async_copy(vals_v.at[pl.ds(_SV + c * _CHUNK, _CHUNK)],
                          out_s.at[pos_b], sb).wait()

  @pl.loop(0, _NCH // 2)
  def _(h):
    c0 = 2 * h
    c1 = 2 * h + 1

    @pl.when(h > 0)
    def _():
      _drain(c0 - 2, pos_a0, pos_b0, sem_a0, sem_b0)

    _chunk(c0, pos_a0, pos_b0)
    _fire(c0, pos_a0, pos_b0, sem_a0, sem_b0)

    @pl.when(h > 0)
    def _():
      _drain(c1 - 2, pos_a1, pos_b1, sem_a1, sem_b1)

    _chunk(c1, pos_a1, pos_b1)
    _fire(c1, pos_a1, pos_b1, sem_a1, sem_b1)

  _drain(_NCH - 2, pos_a0, pos_b0, sem_a0, sem_b0)
  _drain(_NCH - 1, pos_a1, pos_b1, sem_a1, sem_b1)

  pa = keys_v[pl.ds(_NCH * _CHUNK, 16)]
  pb = keys_v[pl.ds(_SV + _NCH * _CHUNK, 16)]
  pos_ta[...] = plsc.load_gather(hist_a, [pa & mask14]) + (pa >> 14)
  pos_tb[...] = plsc.load_gather(hist_b, [pb & mask14]) + (pb >> 14)
  pltpu.sync_copy(vals_v.at[pl.ds(_NCH * _CHUNK, 16)], out_s.at[pos_ta])
  pltpu.sync_copy(vals_v.at[pl.ds(_SV + _NCH * _CHUNK, 16)],
                  out_s.at[pos_tb])
  plsc.subcore_barrier()

  pltpu.sync_copy(out_s.at[pl.ds(wid * _S, _S)], keys_v)
  pltpu.sync_copy(keys_v, out_hbm.at[pl.ds(wid * _S, _S)])


_sort = pl.kernel(
    _body,
    out_type=jax.ShapeDtypeStruct((_N,), jnp.int32),
    mesh=plsc.VectorSubcoreMesh(
        core_axis_name="c", subcore_axis_name="s", num_cores=1),
    compiler_params=pltpu.CompilerParams(needs_layout_passes=False),
    scratch_types=[
        pltpu.VMEM((_S,), jnp.int32),
        pltpu.VMEM((_S,), jnp.int32),
        pltpu.VMEM((_NB,), jnp.int32),
        pltpu.VMEM((_NB,), jnp.int32),
        pltpu.VMEM((_NV * _BR,), jnp.int32),
        pltpu.VMEM((_BR,), jnp.int32),
        pltpu.VMEM((_BR,), jnp.int32),
        pltpu.VMEM((16,), jnp.int32),
        pltpu.VMEM((_CHUNK,), jnp.int32),
        pltpu.VMEM((_CHUNK,), jnp.int32),
        pltpu.VMEM((_CHUNK,), jnp.int32),
        pltpu.VMEM((_CHUNK,), jnp.int32),
        pltpu.VMEM((16,), jnp.int32),
        pltpu.VMEM((16,), jnp.int32),
        pltpu.SMEM((1,), jnp.int32),
        pltpu.VMEM_SHARED((_NV * _NB,), jnp.int32),
        pltpu.VMEM_SHARED((_NW * 16,), jnp.int32),
        pltpu.VMEM_SHARED((_N,), jnp.int32),
        pltpu.SemaphoreType.DMA,
        pltpu.SemaphoreType.DMA,
        pltpu.SemaphoreType.DMA,
        pltpu.SemaphoreType.DMA,
        pltpu.SemaphoreType.DMA,
        pltpu.SemaphoreType.DMA,
        pltpu.SemaphoreType.DMA,
    ],
)


@jax.jit
def kernel(edges, nodes):
  e = edges.astype(jnp.int32)
  flat = _sort(e[:, 0], e[:, 1])
  return flat.reshape(nodes.shape[0], -1)
